# baseline (speedup 1.0000x reference)
import jax
import jax.numpy as jnp
from jax.experimental import pallas as pl
from jax.experimental.pallas import tpu as pltpu

_B = 10000
_S = 2
_C = 2500


def _mlp_block(xa_ref, xb_ref, w1_ref, b1_ref, w2_ref, b2_ref, o_ref):
    b1 = b1_ref[...].astype(jnp.bfloat16)
    for k, x_ref in enumerate((xa_ref, xb_ref)):
        for j in range(_B // _C):
            r0 = j * _C
            h = jnp.dot(
                x_ref[r0 : r0 + _C, :].astype(jnp.bfloat16),
                w1_ref[...],
                preferred_element_type=jnp.float32,
            )
            h = jnp.maximum(h.astype(jnp.bfloat16) + b1, 0)
            ot = jax.lax.dot_general(
                w2_ref[...],
                h,
                dimension_numbers=(((0,), (1,)), ((), ())),
                preferred_element_type=jnp.float32,
            )
            o_ref[k, :, r0 : r0 + _C] = ot[:2, :] + b2_ref[...]


def kernel(x, W1, b1, W2, b2):
    n, d_in = x.shape
    d_hid = W1.shape[1]
    n_cls = W2.shape[1]
    W1 = W1.astype(jnp.bfloat16)
    b1 = b1.reshape(1, d_hid)
    W2p = jnp.pad(W2, ((0, 0), (0, 8 - n_cls))).astype(jnp.bfloat16)
    b2 = b2.reshape(n_cls, 1)
    nb = n // _B
    grid = (nb // _S,)
    ot = pl.pallas_call(
        _mlp_block,
        grid=grid,
        in_specs=[
            pl.BlockSpec((_B, d_in), lambda i: (2 * i, 0)),
            pl.BlockSpec((_B, d_in), lambda i: (2 * i + 1, 0)),
            pl.BlockSpec((d_in, d_hid), lambda i: (0, 0)),
            pl.BlockSpec((1, d_hid), lambda i: (0, 0)),
            pl.BlockSpec((d_hid, 8), lambda i: (0, 0)),
            pl.BlockSpec((n_cls, 1), lambda i: (0, 0)),
        ],
        out_specs=pl.BlockSpec((_S, n_cls, _B), lambda i: (i, 0, 0)),
        out_shape=jax.ShapeDtypeStruct((nb, n_cls, _B), jnp.float32),
        compiler_params=pltpu.CompilerParams(
            dimension_semantics=("parallel",),
        ),
    )(x, x, W1, b1, W2p, b2)
    return ot.transpose(0, 2, 1).reshape(n, n_cls)


# speedup vs baseline: 1.1047x; 1.1047x over previous
import jax
import jax.numpy as jnp
from jax.experimental import pallas as pl
from jax.experimental.pallas import tpu as pltpu

_B = 10000
_S = 2


def _mlp_block(xa_ref, xb_ref, w1_ref, b1_ref, w2_ref, b2_ref, o_ref):
    for k, x_ref in enumerate((xa_ref, xb_ref)):
        h = jnp.dot(
            x_ref[...].astype(jnp.bfloat16),
            w1_ref[...],
            preferred_element_type=jnp.float32,
        )
        h = jnp.maximum(h + b1_ref[...], 0).astype(jnp.bfloat16)
        ot = jax.lax.dot_general(
            w2_ref[...],
            h,
            dimension_numbers=(((0,), (1,)), ((), ())),
            preferred_element_type=jnp.float32,
        )
        o_ref[k] = ot[:2, :] + b2_ref[...]


def kernel(x, W1, b1, W2, b2):
    n, d_in = x.shape
    d_hid = W1.shape[1]
    n_cls = W2.shape[1]
    W1 = W1.astype(jnp.bfloat16)
    b1 = b1.reshape(1, d_hid)
    W2p = jnp.pad(W2, ((0, 0), (0, 8 - n_cls))).astype(jnp.bfloat16)
    b2 = b2.reshape(n_cls, 1)
    nb = n // _B
    grid = (nb // _S,)
    ot = pl.pallas_call(
        _mlp_block,
        grid=grid,
        in_specs=[
            pl.BlockSpec((_B, d_in), lambda i: (2 * i, 0)),
            pl.BlockSpec((_B, d_in), lambda i: (2 * i + 1, 0)),
            pl.BlockSpec((d_in, d_hid), lambda i: (0, 0)),
            pl.BlockSpec((1, d_hid), lambda i: (0, 0)),
            pl.BlockSpec((d_hid, 8), lambda i: (0, 0)),
            pl.BlockSpec((n_cls, 1), lambda i: (0, 0)),
        ],
        out_specs=pl.BlockSpec((_S, n_cls, _B), lambda i: (i, 0, 0)),
        out_shape=jax.ShapeDtypeStruct((nb, n_cls, _B), jnp.float32),
        compiler_params=pltpu.CompilerParams(
            dimension_semantics=("parallel",),
        ),
    )(x, x, W1, b1, W2p, b2)
    return ot.transpose(0, 2, 1).reshape(n, n_cls)
